# baseline (speedup 1.0000x reference)
import functools

import jax
import jax.numpy as jnp
from jax import lax
from jax.experimental import pallas as pl
from jax.experimental.pallas import tpu as pltpu
from jax.experimental.pallas import tpu_sc as plsc

N_E = 8192
E_DIM = 64
GROUPS = 4
NG = N_E // GROUPS
ROWS = 16 * 32 * 32
T = 512
NT = ROWS // T


def _vq_body(z_ref, emb_ref, idx_ref, gidx_ref, me_ref, perp_ref, en_ref,
             counts_ref, esq_ref):
    i = pl.program_id(0)

    @pl.when(i == 0)
    def _init():
        counts_ref[...] = jnp.zeros_like(counts_ref)
        emb = emb_ref[...]
        for g in range(GROUPS):
            e = emb[g * NG:(g + 1) * NG, :]
            en = e / jnp.maximum(
                jnp.sqrt(jnp.sum(e * e, axis=1, keepdims=True)), 1e-12)
            en_ref[g * NG:(g + 1) * NG, :] = en
            esq = jnp.sum(en * en, axis=1, keepdims=True)
            esq_ref[g:g + 1, :] = esq.reshape(1, NG)

    for g in range(GROUPS):
        z = z_ref[:, g * E_DIM:(g + 1) * E_DIM]
        zn = z / jnp.maximum(
            jnp.sqrt(jnp.sum(z * z, axis=1, keepdims=True)), 1e-12)
        en = en_ref[g * NG:(g + 1) * NG, :]

        s = jax.lax.dot_general(zn, en, (((1,), (1,)), ((), ())),
                                preferred_element_type=jnp.float32)
        zsq = jnp.sum(zn * zn, axis=1, keepdims=True)
        d = -zsq - esq_ref[g:g + 1, :] + 2.0 * s

        m = jnp.max(d, axis=1, keepdims=True)
        lane = jax.lax.broadcasted_iota(jnp.int32, (T, NG), 1)
        idx = jnp.max(jnp.where(d >= m, lane, -1), axis=1, keepdims=True)

        idx_ref[:, g:g + 1] = idx
        gidx_ref[:, g:g + 1] = idx + g * NG
        if g == GROUPS - 1:
            oh = (lane == idx).astype(jnp.float32)
            me_ref[...] = oh
            counts_ref[...] += jnp.sum(oh, axis=0, keepdims=True)

    @pl.when(i == NT - 1)
    def _finish():
        avg = counts_ref[...] / float(ROWS)
        ent = jnp.sum(avg * jnp.log(avg + 1e-10), axis=1, keepdims=True)
        perp_ref[...] = jnp.exp(-ent)


_vq_call = pl.pallas_call(
    _vq_body,
    grid=(NT,),
    in_specs=[
        pl.BlockSpec((T, GROUPS * E_DIM), lambda i: (i, 0)),
        pl.BlockSpec((N_E, E_DIM), lambda i: (0, 0)),
    ],
    out_specs=[
        pl.BlockSpec((T, GROUPS), lambda i: (i, 0)),
        pl.BlockSpec((T, GROUPS), lambda i: (i, 0)),
        pl.BlockSpec((T, NG), lambda i: (i, 0)),
        pl.BlockSpec((1, 1), lambda i: (0, 0)),
        pl.BlockSpec((N_E, E_DIM), lambda i: (0, 0)),
    ],
    out_shape=[
        jax.ShapeDtypeStruct((ROWS, GROUPS), jnp.int32),
        jax.ShapeDtypeStruct((ROWS, GROUPS), jnp.int32),
        jax.ShapeDtypeStruct((ROWS, NG), jnp.float32),
        jax.ShapeDtypeStruct((1, 1), jnp.float32),
        jax.ShapeDtypeStruct((N_E, E_DIM), jnp.float32),
    ],
    scratch_shapes=[
        pltpu.VMEM((1, NG), jnp.float32),
        pltpu.VMEM((GROUPS, NG), jnp.float32),
    ],
)

_SC_INFO = plsc.get_sparse_core_info()
_NC = _SC_INFO.num_cores
_NS = _SC_INFO.num_subcores
_L = _SC_INFO.num_lanes
_NW = _NC * _NS


_FPW = (ROWS * GROUPS) // _NW
_NCH = 2
_CH = _FPW // _NCH


@functools.partial(
    pl.kernel,
    mesh=plsc.VectorSubcoreMesh(core_axis_name="c", subcore_axis_name="s"),
    compiler_params=pltpu.CompilerParams(use_tc_tiling_on_sc=False),
    out_type=jax.ShapeDtypeStruct((ROWS * GROUPS, E_DIM), jnp.float32),
    scratch_types=[
        pltpu.VMEM((_CH,), jnp.int32),
        pltpu.VMEM((_CH, E_DIM), jnp.float32),
        pltpu.SemaphoreType.DMA,
    ],
)
def _sc_gather(en_hbm, gidx_hbm, out_hbm, idx_v, rows_v, sem):
    wid = lax.axis_index("s") * _NC + lax.axis_index("c")
    base = wid * _FPW
    for c in range(_NCH):
        fb = base + c * _CH
        pltpu.sync_copy(gidx_hbm.at[pl.ds(fb, _CH)], idx_v)
        pltpu.async_copy(en_hbm.at[idx_v], rows_v, sem).wait()
        pltpu.sync_copy(rows_v, out_hbm.at[pl.ds(fb, _CH), :])


def kernel(z_groups, embedding_weight):
    b = z_groups.shape[0]
    z2d = z_groups.transpose(0, 2, 3, 1).reshape(ROWS, GROUPS * E_DIM)
    idx, gidx, me, perp, en = _vq_call(z2d, embedding_weight)
    zq = _sc_gather(en, gidx.reshape(ROWS * GROUPS))
    quant = (zq.reshape(b, 32, 32, GROUPS * E_DIM)
             .transpose(0, 3, 1, 2))
    zeros_g = jnp.zeros((GROUPS,), jnp.float32)
    return (quant, zeros_g, zeros_g, zeros_g, perp[0, 0], me, idx)


# speedup vs baseline: 1.0780x; 1.0780x over previous
import functools

import jax
import jax.numpy as jnp
from jax import lax
from jax.experimental import pallas as pl
from jax.experimental.pallas import tpu as pltpu
from jax.experimental.pallas import tpu_sc as plsc

N_E = 8192
E_DIM = 64
GROUPS = 4
NG = N_E // GROUPS
ROWS = 16 * 32 * 32
T = 512
NT = ROWS // T


def _vq_body(z_ref, emb_ref, idx_ref, gidx_ref, me_ref, perp_ref, en_ref,
             counts_ref, esq_ref):
    i = pl.program_id(0)

    @pl.when(i == 0)
    def _init():
        counts_ref[...] = jnp.zeros_like(counts_ref)
        emb = emb_ref[...]
        for g in range(GROUPS):
            e = emb[g * NG:(g + 1) * NG, :]
            en = e / jnp.maximum(
                jnp.sqrt(jnp.sum(e * e, axis=1, keepdims=True)), 1e-12)
            en_ref[g * NG:(g + 1) * NG, :] = en
            esq = jnp.sum(en * en, axis=1, keepdims=True)
            esq_ref[g:g + 1, :] = esq.reshape(1, NG)

    for g in range(GROUPS):
        z = z_ref[:, g * E_DIM:(g + 1) * E_DIM]
        zn = z / jnp.maximum(
            jnp.sqrt(jnp.sum(z * z, axis=1, keepdims=True)), 1e-12)
        en = en_ref[g * NG:(g + 1) * NG, :]

        s2 = jax.lax.dot_general(zn + zn, en, (((1,), (1,)), ((), ())),
                                 preferred_element_type=jnp.float32)
        zsq = jnp.sum(zn * zn, axis=1, keepdims=True)
        d = -zsq - esq_ref[g:g + 1, :] + s2

        m = jnp.max(d, axis=1, keepdims=True)
        lane = jax.lax.broadcasted_iota(jnp.int32, (T, NG), 1)
        idx = jnp.max(jnp.where(d >= m, lane, -1), axis=1, keepdims=True)

        idx_ref[:, g:g + 1] = idx
        gidx_ref[:, g:g + 1] = idx + g * NG
        if g == GROUPS - 1:
            oh = (lane == idx).astype(jnp.float32)
            me_ref[...] = oh
            counts_ref[...] += jnp.sum(oh, axis=0, keepdims=True)

    @pl.when(i == NT - 1)
    def _finish():
        avg = counts_ref[...] / float(ROWS)
        ent = jnp.sum(avg * jnp.log(avg + 1e-10), axis=1, keepdims=True)
        perp_ref[...] = jnp.exp(-ent)


_vq_call = pl.pallas_call(
    _vq_body,
    grid=(NT,),
    in_specs=[
        pl.BlockSpec((T, GROUPS * E_DIM), lambda i: (i, 0)),
        pl.BlockSpec((N_E, E_DIM), lambda i: (0, 0)),
    ],
    out_specs=[
        pl.BlockSpec((T, GROUPS), lambda i: (i, 0)),
        pl.BlockSpec((T, GROUPS), lambda i: (i, 0)),
        pl.BlockSpec((T, NG), lambda i: (i, 0)),
        pl.BlockSpec((1, 1), lambda i: (0, 0)),
        pl.BlockSpec((N_E, E_DIM), lambda i: (0, 0)),
    ],
    out_shape=[
        jax.ShapeDtypeStruct((ROWS, GROUPS), jnp.int32),
        jax.ShapeDtypeStruct((ROWS, GROUPS), jnp.int32),
        jax.ShapeDtypeStruct((ROWS, NG), jnp.float32),
        jax.ShapeDtypeStruct((1, 1), jnp.float32),
        jax.ShapeDtypeStruct((N_E, E_DIM), jnp.float32),
    ],
    scratch_shapes=[
        pltpu.VMEM((1, NG), jnp.float32),
        pltpu.VMEM((GROUPS, NG), jnp.float32),
    ],
)

_SC_INFO = plsc.get_sparse_core_info()
_NC = _SC_INFO.num_cores
_NS = _SC_INFO.num_subcores
_L = _SC_INFO.num_lanes
_NW = _NC * _NS


_FPW = (ROWS * GROUPS) // _NW
_NCH = 2
_CH = _FPW // _NCH


@functools.partial(
    pl.kernel,
    mesh=plsc.VectorSubcoreMesh(core_axis_name="c", subcore_axis_name="s"),
    compiler_params=pltpu.CompilerParams(use_tc_tiling_on_sc=False),
    out_type=jax.ShapeDtypeStruct((ROWS * GROUPS, E_DIM), jnp.float32),
    scratch_types=[
        pltpu.VMEM((_CH,), jnp.int32),
        pltpu.VMEM((_CH, E_DIM), jnp.float32),
        pltpu.SemaphoreType.DMA,
    ],
)
def _sc_gather(en_hbm, gidx_hbm, out_hbm, idx_v, rows_v, sem):
    wid = lax.axis_index("s") * _NC + lax.axis_index("c")
    base = wid * _FPW
    for c in range(_NCH):
        fb = base + c * _CH
        pltpu.sync_copy(gidx_hbm.at[pl.ds(fb, _CH)], idx_v)
        pltpu.async_copy(en_hbm.at[idx_v], rows_v, sem).wait()
        pltpu.sync_copy(rows_v, out_hbm.at[pl.ds(fb, _CH), :])


def kernel(z_groups, embedding_weight):
    b = z_groups.shape[0]
    z2d = z_groups.transpose(0, 2, 3, 1).reshape(ROWS, GROUPS * E_DIM)
    idx, gidx, me, perp, en = _vq_call(z2d, embedding_weight)
    zq = _sc_gather(en, gidx.reshape(ROWS * GROUPS))
    quant = (zq.reshape(b, 32, 32, GROUPS * E_DIM)
             .transpose(0, 3, 1, 2))
    zeros_g = jnp.zeros((GROUPS,), jnp.float32)
    return (quant, zeros_g, zeros_g, zeros_g, perp[0, 0], me, idx)


# speedup vs baseline: 1.1622x; 1.0781x over previous
import functools

import jax
import jax.numpy as jnp
from jax import lax
from jax.experimental import pallas as pl
from jax.experimental.pallas import tpu as pltpu
from jax.experimental.pallas import tpu_sc as plsc

N_E = 8192
E_DIM = 64
GROUPS = 4
NG = N_E // GROUPS
ROWS = 16 * 32 * 32
T = 512
NT = ROWS // T


def _vq_body(z_ref, emb_ref, idx_ref, gidx_ref, me_ref, perp_ref, en_ref,
             counts_ref, esq_ref):
    i = pl.program_id(0)

    @pl.when(i == 0)
    def _init():
        counts_ref[...] = jnp.zeros_like(counts_ref)
        emb = emb_ref[...]
        for g in range(GROUPS):
            e = emb[g * NG:(g + 1) * NG, :]
            en = e / jnp.maximum(
                jnp.sqrt(jnp.sum(e * e, axis=1, keepdims=True)), 1e-12)
            en_ref[g * NG:(g + 1) * NG, :] = en
            esq = jnp.sum(en * en, axis=1, keepdims=True)
            esq_ref[g:g + 1, :] = esq.reshape(1, NG)

    for g in range(GROUPS):
        z = z_ref[:, g * E_DIM:(g + 1) * E_DIM]
        zn = z / jnp.maximum(
            jnp.sqrt(jnp.sum(z * z, axis=1, keepdims=True)), 1e-12)
        en = en_ref[g * NG:(g + 1) * NG, :]

        s2 = jax.lax.dot_general(zn + zn, en, (((1,), (1,)), ((), ())),
                                 preferred_element_type=jnp.float32)
        zsq = jnp.sum(zn * zn, axis=1, keepdims=True)
        d = -zsq - esq_ref[g:g + 1, :] + s2

        m = jnp.max(d, axis=1, keepdims=True)
        lane_f = jax.lax.broadcasted_iota(jnp.int32, (T, NG), 1).astype(
            jnp.float32)
        idx_f = jnp.max(jnp.where(d >= m, lane_f, -1.0), axis=1,
                        keepdims=True)
        idx = idx_f.astype(jnp.int32)

        idx_ref[:, g:g + 1] = idx
        gidx_ref[:, g:g + 1] = idx + g * NG
        if g == GROUPS - 1:
            oh = (lane_f == idx_f).astype(jnp.float32)
            me_ref[...] = oh
            counts_ref[...] += jnp.sum(oh, axis=0, keepdims=True)

    @pl.when(i == NT - 1)
    def _finish():
        avg = counts_ref[...] / float(ROWS)
        ent = jnp.sum(avg * jnp.log(avg + 1e-10), axis=1, keepdims=True)
        perp_ref[...] = jnp.exp(-ent)


_vq_call = pl.pallas_call(
    _vq_body,
    grid=(NT,),
    in_specs=[
        pl.BlockSpec((T, GROUPS * E_DIM), lambda i: (i, 0)),
        pl.BlockSpec((N_E, E_DIM), lambda i: (0, 0)),
    ],
    out_specs=[
        pl.BlockSpec((T, GROUPS), lambda i: (i, 0)),
        pl.BlockSpec((T, GROUPS), lambda i: (i, 0)),
        pl.BlockSpec((T, NG), lambda i: (i, 0)),
        pl.BlockSpec((1, 1), lambda i: (0, 0)),
        pl.BlockSpec((N_E, E_DIM), lambda i: (0, 0)),
    ],
    out_shape=[
        jax.ShapeDtypeStruct((ROWS, GROUPS), jnp.int32),
        jax.ShapeDtypeStruct((ROWS, GROUPS), jnp.int32),
        jax.ShapeDtypeStruct((ROWS, NG), jnp.float32),
        jax.ShapeDtypeStruct((1, 1), jnp.float32),
        jax.ShapeDtypeStruct((N_E, E_DIM), jnp.float32),
    ],
    scratch_shapes=[
        pltpu.VMEM((1, NG), jnp.float32),
        pltpu.VMEM((GROUPS, NG), jnp.float32),
    ],
)

_SC_INFO = plsc.get_sparse_core_info()
_NC = _SC_INFO.num_cores
_NS = _SC_INFO.num_subcores
_L = _SC_INFO.num_lanes
_NW = _NC * _NS


_FPW = (ROWS * GROUPS) // _NW
_NCH = 2
_CH = _FPW // _NCH


@functools.partial(
    pl.kernel,
    mesh=plsc.VectorSubcoreMesh(core_axis_name="c", subcore_axis_name="s"),
    compiler_params=pltpu.CompilerParams(use_tc_tiling_on_sc=False),
    out_type=jax.ShapeDtypeStruct((ROWS * GROUPS, E_DIM), jnp.float32),
    scratch_types=[
        pltpu.VMEM((_CH,), jnp.int32),
        pltpu.VMEM((_CH, E_DIM), jnp.float32),
        pltpu.SemaphoreType.DMA,
    ],
)
def _sc_gather(en_hbm, gidx_hbm, out_hbm, idx_v, rows_v, sem):
    wid = lax.axis_index("s") * _NC + lax.axis_index("c")
    base = wid * _FPW
    for c in range(_NCH):
        fb = base + c * _CH
        pltpu.sync_copy(gidx_hbm.at[pl.ds(fb, _CH)], idx_v)
        pltpu.async_copy(en_hbm.at[idx_v], rows_v, sem).wait()
        pltpu.sync_copy(rows_v, out_hbm.at[pl.ds(fb, _CH), :])


def kernel(z_groups, embedding_weight):
    b = z_groups.shape[0]
    z2d = z_groups.transpose(0, 2, 3, 1).reshape(ROWS, GROUPS * E_DIM)
    idx, gidx, me, perp, en = _vq_call(z2d, embedding_weight)
    zq = _sc_gather(en, gidx.reshape(ROWS * GROUPS))
    quant = (zq.reshape(b, 32, 32, GROUPS * E_DIM)
             .transpose(0, 3, 1, 2))
    zeros_g = jnp.zeros((GROUPS,), jnp.float32)
    return (quant, zeros_g, zeros_g, zeros_g, perp[0, 0], me, idx)
